# Initial kernel scaffold; baseline (speedup 1.0000x reference)
import functools

import jax
import jax.numpy as jnp
from jax import lax
from jax.experimental import pallas as pl
from jax.experimental.pallas import tpu as pltpu
from jax.experimental.pallas import tpu_sc as plsc

_BATCH = 4096
_OFF = 20
_D = 64
_NW = 32
_BAGS_W = _BATCH // _NW
_CB = 32
_NCH = _BAGS_W // _CB
_RPC = _CB * _OFF
_G = 128
_NG = _RPC // _G


def _make_kernel():
    mesh = plsc.VectorSubcoreMesh(core_axis_name="c", subcore_axis_name="s")

    @functools.partial(
        pl.kernel,
        mesh=mesh,
        out_type=jax.ShapeDtypeStruct((_BATCH, _D), jnp.float32),
        scratch_types=[
            pltpu.VMEM((_NG, _G), jnp.int32),
            pltpu.VMEM((_RPC, _D), jnp.float32),
            pltpu.VMEM((_CB, _D), jnp.float32),
            pltpu.SemaphoreType.DMA,
        ],
    )
    def emb_bag(table, idx2d, out, idx_v, rows_v, out_v, sem):
        w = lax.axis_index("s") * 2 + lax.axis_index("c")
        for ci in range(_NCH):
            blk0 = w * (_NCH * _NG) + ci * _NG
            pltpu.sync_copy(idx2d.at[pl.ds(blk0, _NG)], idx_v)
            cps = [
                pltpu.async_copy(
                    table.at[idx_v.at[j]],
                    rows_v.at[pl.ds(j * _G, _G)],
                    sem,
                )
                for j in range(_NG)
            ]
            for cp in cps:
                cp.wait()

            def bag_body(b, carry):
                r0 = b * _OFF
                for c in range(_D // 16):
                    acc = rows_v[r0, pl.ds(c * 16, 16)]
                    for r in range(1, _OFF):
                        acc = acc + rows_v[r0 + r, pl.ds(c * 16, 16)]
                    out_v[b, pl.ds(c * 16, 16)] = acc
                return carry

            lax.fori_loop(0, _CB, bag_body, 0)
            bag0 = w * _BAGS_W + ci * _CB
            pltpu.sync_copy(out_v, out.at[pl.ds(bag0, _CB)])

    return emb_bag


_EMB_BAG = _make_kernel()


@jax.jit
def kernel(sparse_index_group_batch, sparse_offset_group_batch, weight):
    del sparse_offset_group_batch
    idx2d = sparse_index_group_batch.astype(jnp.int32).reshape(
        _BATCH * _OFF // _G, _G
    )
    return _EMB_BAG(weight, idx2d)


# speedup vs baseline: 1.3751x; 1.3751x over previous
import functools

import jax
import jax.numpy as jnp
from jax import lax
from jax.experimental import pallas as pl
from jax.experimental.pallas import tpu as pltpu
from jax.experimental.pallas import tpu_sc as plsc

_BATCH = 4096
_OFF = 20
_D = 64
_NW = 32
_BAGS_W = _BATCH // _NW
_CB = 32
_NCH = _BAGS_W // _CB
_RPC = _CB * _OFF
_G = 128
_NG = _RPC // _G


def _make_kernel():
    mesh = plsc.VectorSubcoreMesh(core_axis_name="c", subcore_axis_name="s")

    @functools.partial(
        pl.kernel,
        mesh=mesh,
        out_type=jax.ShapeDtypeStruct((_BATCH, _D), jnp.float32),
        scratch_types=[
            pltpu.VMEM((_NCH * _NG, _G), jnp.int32),
            pltpu.VMEM((_RPC, _D), jnp.float32),
            pltpu.VMEM((_CB, _D), jnp.float32),
            pltpu.SemaphoreType.DMA,
        ],
        compiler_params=pltpu.CompilerParams(use_tc_tiling_on_sc=False),
    )
    def emb_bag(table, idx3d, out, idx_v, rows_v, out_v, sem):
        w = lax.axis_index("s") * 2 + lax.axis_index("c")
        pltpu.sync_copy(idx3d.at[w], idx_v)
        for ci in range(_NCH):
            cps = [
                pltpu.async_copy(
                    table.at[idx_v.at[ci * _NG + j]],
                    rows_v.at[pl.ds(j * _G, _G)],
                    sem,
                )
                for j in range(_NG)
            ]
            for cp in cps:
                cp.wait()

            def bag_body(b, carry):
                r0 = b * _OFF
                for c in range(_D // 16):
                    acc = rows_v[r0, pl.ds(c * 16, 16)]
                    for r in range(1, _OFF):
                        acc = acc + rows_v[r0 + r, pl.ds(c * 16, 16)]
                    out_v[b, pl.ds(c * 16, 16)] = acc
                return carry

            lax.fori_loop(0, _CB, bag_body, 0)
            bag0 = w * _BAGS_W + ci * _CB
            pltpu.sync_copy(out_v, out.at[pl.ds(bag0, _CB)])

    return emb_bag


_EMB_BAG = _make_kernel()


@jax.jit
def kernel(sparse_index_group_batch, sparse_offset_group_batch, weight):
    del sparse_offset_group_batch
    idx3d = sparse_index_group_batch.astype(jnp.int32).reshape(
        _NW, _NCH * _NG, _G
    )
    return _EMB_BAG(weight, idx3d)


# speedup vs baseline: 1.5139x; 1.1010x over previous
import functools

import jax
import jax.numpy as jnp
from jax import lax
from jax.experimental import pallas as pl
from jax.experimental.pallas import tpu as pltpu
from jax.experimental.pallas import tpu_sc as plsc

_BATCH = 4096
_OFF = 20
_D = 64
_NW = 32
_BAGS_W = _BATCH // _NW
_CB = 32
_NCH = _BAGS_W // _CB
_RPC = _CB * _OFF
_G = 128
_NG = _RPC // _G


def _tree_sum(vals):
    while len(vals) > 1:
        nxt = [vals[i] + vals[i + 1] for i in range(0, len(vals) - 1, 2)]
        if len(vals) % 2:
            nxt.append(vals[-1])
        vals = nxt
    return vals[0]


def _make_kernel():
    mesh = plsc.VectorSubcoreMesh(core_axis_name="c", subcore_axis_name="s")

    @functools.partial(
        pl.kernel,
        mesh=mesh,
        out_type=jax.ShapeDtypeStruct((_BATCH, _D), jnp.float32),
        scratch_types=[
            pltpu.VMEM((_NCH * _NG, _G), jnp.int32),
            pltpu.VMEM((_RPC, _D), jnp.float32),
            pltpu.VMEM((_RPC, _D), jnp.float32),
            pltpu.VMEM((_CB, _D), jnp.float32),
            pltpu.VMEM((_CB, _D), jnp.float32),
            pltpu.SemaphoreType.DMA,
            pltpu.SemaphoreType.DMA,
            pltpu.SemaphoreType.DMA,
            pltpu.SemaphoreType.DMA,
        ],
        compiler_params=pltpu.CompilerParams(use_tc_tiling_on_sc=False),
    )
    def emb_bag(table, idx3d, out, idx_v, rows0, rows1, out0, out1,
                gsem0, gsem1, osem0, osem1):
        rows = (rows0, rows1)
        outb = (out0, out1)
        gsem = (gsem0, gsem1)
        osem = (osem0, osem1)

        w = lax.axis_index("s") * 2 + lax.axis_index("c")
        pltpu.sync_copy(idx3d.at[w], idx_v)

        def fire(ci):
            buf, sem = rows[ci % 2], gsem[ci % 2]
            return [
                pltpu.async_copy(
                    table.at[idx_v.at[ci * _NG + j]],
                    buf.at[pl.ds(j * _G, _G)],
                    sem,
                )
                for j in range(_NG)
            ]

        out_cp = [None, None]
        pending = fire(0)
        for ci in range(_NCH):
            nxt = fire(ci + 1) if ci + 1 < _NCH else []
            for cp in pending:
                cp.wait()
            pending = nxt

            buf = rows[ci % 2]
            ob = outb[ci % 2]
            if out_cp[ci % 2] is not None:
                out_cp[ci % 2].wait()

            def bag_body(b, carry, buf=buf, ob=ob):
                r0 = b * _OFF
                for c in range(_D // 16):
                    vals = [
                        buf[r0 + r, pl.ds(c * 16, 16)] for r in range(_OFF)
                    ]
                    ob[b, pl.ds(c * 16, 16)] = _tree_sum(vals)
                return carry

            lax.fori_loop(0, _CB, bag_body, 0, unroll=2)

            bag0 = w * _BAGS_W + ci * _CB
            out_cp[ci % 2] = pltpu.async_copy(
                ob, out.at[pl.ds(bag0, _CB)], osem[ci % 2]
            )
        for cp in out_cp:
            if cp is not None:
                cp.wait()

    return emb_bag


_EMB_BAG = _make_kernel()


@jax.jit
def kernel(sparse_index_group_batch, sparse_offset_group_batch, weight):
    del sparse_offset_group_batch
    idx3d = sparse_index_group_batch.astype(jnp.int32).reshape(
        _NW, _NCH * _NG, _G
    )
    return _EMB_BAG(weight, idx3d)
